# Initial kernel scaffold; baseline (speedup 1.0000x reference)
import jax
import jax.numpy as jnp
from jax.experimental import pallas as pl


def kernel(x, W0, g0, b0, W1, g1, b1, W2, g2, b2, W3, g3, b3, W4, g4, b4):
    raise NotImplementedError("write your pallas kernel here")


# speedup vs baseline: 4.7599x; 4.7599x over previous
import functools

import jax
import jax.numpy as jnp
from jax import lax
from jax.experimental import pallas as pl
from jax.experimental.pallas import tpu as pltpu
from jax.experimental.pallas import tpu_sc as plsc

K = 20
KP = 24
B = 8
N = 1024
ROWS = 256
NB = (B * N) // ROWS
NTILES = 32
PPT = (B * N) // NTILES
EPS = 1e-5
NEG_SLOPE = 0.2


def _topk_body(xr_ref, xa_ref, idx_ref):
    b = pl.program_id(0)
    xr = xr_ref[0]
    xa = xa_ref[0]
    cn = (((1,), (1,)), ((), ()))
    g = lax.dot_general(xr, xa, cn, preferred_element_type=jnp.float32)
    inner = -2.0 * g
    xxr = jnp.sum(xr * xr, axis=-1, keepdims=True)
    xxa = jnp.sum(xa * xa, axis=-1, keepdims=True)
    a = (-xxr) - inner - jnp.transpose(xxa)
    iota = lax.broadcasted_iota(jnp.int32, (ROWS, N), 1)
    cols = []
    for _ in range(K):
        m = jnp.max(a, axis=1, keepdims=True)
        cand = jnp.where(a >= m, iota, N)
        j = jnp.min(cand, axis=1, keepdims=True)
        cols.append(j)
        a = jnp.where(iota == j, -jnp.inf, a)
    idx = jnp.concatenate(cols + [cols[0]] * (KP - K), axis=1)
    idx_ref[0] = idx + b * N


def _tc_topk(xt):
    _, _, C = xt.shape
    grid = (B, N // ROWS)
    return pl.pallas_call(
        _topk_body,
        grid=grid,
        in_specs=[
            pl.BlockSpec((1, ROWS, C), lambda b, r: (b, r, 0)),
            pl.BlockSpec((1, N, C), lambda b, r: (b, 0, 0)),
        ],
        out_specs=pl.BlockSpec((1, ROWS, KP), lambda b, r: (b, r, 0)),
        out_shape=jax.ShapeDtypeStruct((B, N, KP), jnp.int32),
    )(xt, xt)


def _sc_gather_body(x_hbm, idx_hbm, out_hbm, idx_v, rows_v, gsem):
    cid = lax.axis_index("c")
    sid = lax.axis_index("s")
    wid = sid * 2 + cid
    base = wid * PPT
    pltpu.sync_copy(idx_hbm.at[pl.ds(base, PPT)], idx_v)

    def body(i, _):
        gcp = pltpu.async_copy(x_hbm.at[idx_v.at[i]], rows_v, gsem)
        gcp.wait()
        pltpu.sync_copy(rows_v, out_hbm.at[pl.ds((base + i) * KP, KP)])
        return 0

    lax.fori_loop(0, PPT, body, 0, unroll=False)


def _sc_gather(x2d, idx2d):
    C = x2d.shape[1]
    mesh = plsc.VectorSubcoreMesh(core_axis_name="c", subcore_axis_name="s")
    f = pl.kernel(
        _sc_gather_body,
        out_type=jax.ShapeDtypeStruct((B * N * KP, C), jnp.float32),
        mesh=mesh,
        scratch_types=[
            pltpu.VMEM((PPT, KP), jnp.int32),
            pltpu.VMEM((KP, C), jnp.float32),
            pltpu.SemaphoreType.DMA,
        ],
    )
    return f(x2d, idx2d)


def _conv_body(C, O, x_ref, n_ref, w_ref, m_ref, part_ref):
    xc = x_ref[...]
    w = w_ref[...]
    cn = (((1,), (0,)), ((), ()))
    mx = None
    s1 = None
    s2 = None
    for k in range(K):
        nk = n_ref[:, pl.ds(k * C, C)]
        ef = jnp.concatenate([nk - xc, xc], axis=1)
        h = lax.dot_general(ef, w, cn, preferred_element_type=jnp.float32)
        if k == 0:
            mx = h
            s1 = h
            s2 = h * h
        else:
            mx = jnp.maximum(mx, h)
            s1 = s1 + h
            s2 = s2 + h * h
    m_ref[...] = mx
    part_ref[0, 0] = jnp.sum(s1, axis=0)
    part_ref[0, 1] = jnp.sum(s2, axis=0)


def _tc_conv(x2d, neigh, w2):
    C = x2d.shape[1]
    O = w2.shape[1]
    grid = (NB,)
    return pl.pallas_call(
        functools.partial(_conv_body, C, O),
        grid=grid,
        in_specs=[
            pl.BlockSpec((ROWS, C), lambda r: (r, 0)),
            pl.BlockSpec((ROWS, KP * C), lambda r: (r, 0)),
            pl.BlockSpec((2 * C, O), lambda r: (0, 0)),
        ],
        out_specs=[
            pl.BlockSpec((ROWS, O), lambda r: (r, 0)),
            pl.BlockSpec((1, 2, O), lambda r: (r, 0, 0)),
        ],
        out_shape=[
            jax.ShapeDtypeStruct((B * N, O), jnp.float32),
            jax.ShapeDtypeStruct((NB, 2, O), jnp.float32),
        ],
    )(x2d, neigh, w2)


def _norm_body(Op, g_ref, b_ref, part_ref, m_ref, o_ref):
    inv_cnt = 1.0 / float(B * N * K)
    s1 = jnp.sum(part_ref[:, 0], axis=0, keepdims=True) * inv_cnt
    s2 = jnp.sum(part_ref[:, 1], axis=0, keepdims=True) * inv_cnt
    var = s2 - s1 * s1
    h = (m_ref[...] - s1) / jnp.sqrt(var + EPS) * g_ref[...] + b_ref[...]
    out = jnp.where(h > 0, h, NEG_SLOPE * h)
    o_ref[...] = jnp.pad(out, ((0, 0), (0, Op - out.shape[1])))


def _tc_norm(m2d, part, gl, bl, Op):
    O = m2d.shape[1]
    rows = 1024
    grid = ((B * N) // rows,)
    return pl.pallas_call(
        functools.partial(_norm_body, Op),
        grid=grid,
        in_specs=[
            pl.BlockSpec((1, O), lambda r: (0, 0)),
            pl.BlockSpec((1, O), lambda r: (0, 0)),
            pl.BlockSpec((NB, 2, O), lambda r: (0, 0, 0)),
            pl.BlockSpec((rows, O), lambda r: (r, 0)),
        ],
        out_specs=pl.BlockSpec((rows, Op), lambda r: (r, 0)),
        out_shape=jax.ShapeDtypeStruct((B * N, Op), jnp.float32),
    )(gl.reshape(1, O), bl.reshape(1, O), part, m2d)


def _edge_conv_layer(x2d, W, gl, bl, Op):
    Cp = x2d.shape[1]
    O = W.shape[0]
    C = W.shape[1] // 2
    wa = jnp.transpose(W[:, :C])
    wb = jnp.transpose(W[:, C:])
    z = jnp.zeros((Cp - C, O), jnp.float32)
    w2 = jnp.concatenate([wa, z, wb, z], axis=0)
    idx = _tc_topk(x2d.reshape(B, N, Cp))
    neigh = _sc_gather(x2d, idx.reshape(B * N, KP))
    m2d, part = _tc_conv(x2d, neigh.reshape(B * N, KP * Cp), w2)
    return _tc_norm(m2d, part, gl, bl, Op)


def kernel(x, W0, g0, b0, W1, g1, b1, W2, g2, b2, W3, g3, b3, W4, g4, b4):
    xt = jnp.transpose(x, (0, 2, 1)).reshape(B * N, 3)
    x2d = jnp.pad(xt, ((0, 0), (0, 125)))
    outs = []
    for W, g, bb in ((W0, g0, b0), (W1, g1, b1), (W2, g2, b2),
                     (W3, g3, b3), (W4, g4, b4)):
        O = W.shape[0]
        Op = max(O, 128)
        x2d = _edge_conv_layer(x2d, W, g, bb, Op)
        outs.append(x2d[:, :O].reshape(B, N, O))
    y = jnp.concatenate(outs, axis=-1)
    return jnp.transpose(y, (0, 2, 1))
